# baseline (speedup 1.0000x reference)
import functools

import jax
import jax.numpy as jnp
from jax import lax
from jax.experimental import pallas as pl
from jax.experimental.pallas import tpu as pltpu
from jax.experimental.pallas import tpu_sc as plsc

N_TOK = 36864
DIM = 64
K_EMB = 8192
COMMIT_SCALE = 1.25

BN = 512
KC = 2048


def _tc_body(x_ref, emb_ref, idx_ref, loss_ref, en_ref, en_scr, s_scr, acc_ref):
    i = pl.program_id(0)
    nblk = pl.num_programs(0)

    @pl.when(i == 0)
    def _init():
        e = emb_ref[...]
        en = e / jnp.maximum(
            jnp.sqrt(jnp.sum(e * e, axis=1, keepdims=True)), 1e-12)
        en_scr[...] = en.T
        en_ref[...] = en
        s_scr[...] = jnp.sum(en * en, axis=1)[None, :]
        acc_ref[0] = 0.0

    xb = x_ref[...]
    xn = xb / jnp.maximum(
        jnp.sqrt(jnp.sum(xb * xb, axis=1, keepdims=True)), 1e-12)
    c = jnp.sum(xn * xn, axis=1, keepdims=True)
    m_run = jnp.full((BN, 1), jnp.inf, dtype=jnp.float32)
    idx_run = jnp.full((BN, 1), K_EMB, dtype=jnp.int32)
    for j in range(K_EMB // KC):
        dot = lax.dot_general(xn, en_scr[:, j * KC:(j + 1) * KC],
                              (((1,), (0,)), ((), ())),
                              preferred_element_type=jnp.float32)
        dist = c + s_scr[:, j * KC:(j + 1) * KC] - 2.0 * dot
        m = jnp.min(dist, axis=1, keepdims=True)
        kio = lax.broadcasted_iota(jnp.int32, dist.shape, 1) + j * KC
        idx = jnp.min(jnp.where(dist == m, kio, K_EMB), axis=1,
                      keepdims=True)
        upd = m < m_run
        idx_run = jnp.where(upd, idx, idx_run)
        m_run = jnp.where(upd, m, m_run)
    idx_ref[0, 0, :] = idx_run[:, 0]
    acc_ref[0] += jnp.sum(m_run)

    @pl.when(i == nblk - 1)
    def _fin():
        loss_ref[0, 0] = acc_ref[0] * (COMMIT_SCALE / (N_TOK * DIM))


def _distance_argmin(x, embeddings):
    nblk = N_TOK // BN
    return pl.pallas_call(
        _tc_body,
        grid=(nblk,),
        in_specs=[
            pl.BlockSpec((BN, DIM), lambda i: (i, 0)),
            pl.BlockSpec((K_EMB, DIM), lambda i: (0, 0)),
        ],
        out_specs=[
            pl.BlockSpec((1, 1, BN), lambda i: (i, 0, 0)),
            pl.BlockSpec(memory_space=pltpu.SMEM),
            pl.BlockSpec((K_EMB, DIM), lambda i: (0, 0)),
        ],
        out_shape=[
            jax.ShapeDtypeStruct((nblk, 1, BN), jnp.int32),
            jax.ShapeDtypeStruct((1, 1), jnp.float32),
            jax.ShapeDtypeStruct((K_EMB, DIM), jnp.float32),
        ],
        scratch_shapes=[
            pltpu.VMEM((DIM, K_EMB), jnp.float32),
            pltpu.VMEM((1, K_EMB), jnp.float32),
            pltpu.SMEM((1,), jnp.float32),
        ],
    )(x, embeddings)


def _sc_gather(table, idx):
    info = plsc.get_sparse_core_info()
    nw = info.num_cores * info.num_subcores
    b_per_w = N_TOK // nw
    mesh = plsc.VectorSubcoreMesh(core_axis_name="c", subcore_axis_name="s")

    @functools.partial(
        pl.kernel,
        out_type=jax.ShapeDtypeStruct((N_TOK, DIM), jnp.float32),
        mesh=mesh,
        compiler_params=pltpu.CompilerParams(use_tc_tiling_on_sc=False),
        scratch_types=[
            pltpu.VMEM((b_per_w,), jnp.int32),
            pltpu.VMEM((b_per_w, DIM), jnp.float32),
            pltpu.SemaphoreType.DMA,
        ],
    )
    def gather(table_hbm, idx_hbm, out_hbm, idx_v, rows_v, sem):
        wid = lax.axis_index("s") * info.num_cores + lax.axis_index("c")
        base = wid * b_per_w
        pltpu.sync_copy(idx_hbm.at[pl.ds(base, b_per_w)], idx_v)
        pltpu.async_copy(table_hbm.at[idx_v], rows_v, sem).wait()
        pltpu.sync_copy(rows_v, out_hbm.at[pl.ds(base, b_per_w)])

    return gather(table, idx)


def kernel(x, embeddings):
    idx3, loss2, e_n = _distance_argmin(x, embeddings)
    indices = idx3.reshape(N_TOK)
    quantized = _sc_gather(e_n, indices)
    return (quantized, loss2[0, 0], indices)


# speedup vs baseline: 1.0172x; 1.0172x over previous
import functools

import jax
import jax.numpy as jnp
from jax import lax
from jax.experimental import pallas as pl
from jax.experimental.pallas import tpu as pltpu
from jax.experimental.pallas import tpu_sc as plsc

N_TOK = 36864
DIM = 64
K_EMB = 8192
COMMIT_SCALE = 1.25

BN = 512


def _prep_body(emb_ref, en_ref, ent_ref, s_ref):
    e = emb_ref[...]
    en = e / jnp.maximum(
        jnp.sqrt(jnp.sum(e * e, axis=1, keepdims=True)), 1e-12)
    en_ref[...] = en
    ent_ref[...] = en.T
    s_ref[...] = jnp.sum(en * en, axis=1)[None, :]


def _prep(embeddings):
    return pl.pallas_call(
        _prep_body,
        out_shape=[
            jax.ShapeDtypeStruct((K_EMB, DIM), jnp.float32),
            jax.ShapeDtypeStruct((DIM, K_EMB), jnp.float32),
            jax.ShapeDtypeStruct((1, K_EMB), jnp.float32),
        ],
    )(embeddings)


def _main_body(x_ref, ent_ref, s_ref, idx_ref, loss_ref, acc_ref):
    i = pl.program_id(0)
    nblk = pl.num_programs(0)

    @pl.when(i == 0)
    def _init():
        acc_ref[0] = 0.0

    xb = x_ref[...]
    xn = xb / jnp.maximum(
        jnp.sqrt(jnp.sum(xb * xb, axis=1, keepdims=True)), 1e-12)
    c = jnp.sum(xn * xn, axis=1, keepdims=True)
    dot = lax.dot_general(xn, ent_ref[...], (((1,), (0,)), ((), ())),
                          preferred_element_type=jnp.float32)
    dist = c + s_ref[...] - 2.0 * dot
    m = jnp.min(dist, axis=1, keepdims=True)
    kiota = lax.broadcasted_iota(jnp.int32, dist.shape, 1)
    idx_ref[0, 0, :] = jnp.min(jnp.where(dist == m, kiota, K_EMB), axis=1)
    acc_ref[0] += jnp.sum(m)

    @pl.when(i == nblk - 1)
    def _fin():
        loss_ref[0, 0] = acc_ref[0]


def _distance_argmin(xh, ent, s):
    nh = xh.shape[0]
    nblk = nh // BN
    return pl.pallas_call(
        _main_body,
        grid=(nblk,),
        in_specs=[
            pl.BlockSpec((BN, DIM), lambda i: (i, 0)),
            pl.BlockSpec((DIM, K_EMB), lambda i: (0, 0)),
            pl.BlockSpec((1, K_EMB), lambda i: (0, 0)),
        ],
        out_specs=[
            pl.BlockSpec((1, 1, BN), lambda i: (i, 0, 0)),
            pl.BlockSpec(memory_space=pltpu.SMEM),
        ],
        out_shape=[
            jax.ShapeDtypeStruct((nblk, 1, BN), jnp.int32),
            jax.ShapeDtypeStruct((1, 1), jnp.float32),
        ],
        scratch_shapes=[
            pltpu.SMEM((1,), jnp.float32),
        ],
    )(xh, ent, s)


def _sc_gather(table, idx):
    n = idx.shape[0]
    info = plsc.get_sparse_core_info()
    nw = info.num_cores * info.num_subcores
    b_per_w = n // nw
    mesh = plsc.VectorSubcoreMesh(core_axis_name="c", subcore_axis_name="s")

    @functools.partial(
        pl.kernel,
        out_type=jax.ShapeDtypeStruct((n, DIM), jnp.float32),
        mesh=mesh,
        compiler_params=pltpu.CompilerParams(use_tc_tiling_on_sc=False),
        scratch_types=[
            pltpu.VMEM((b_per_w,), jnp.int32),
            pltpu.VMEM((b_per_w, DIM), jnp.float32),
            pltpu.SemaphoreType.DMA,
        ],
    )
    def gather(table_hbm, idx_hbm, out_hbm, idx_v, rows_v, sem):
        wid = lax.axis_index("s") * info.num_cores + lax.axis_index("c")
        base = wid * b_per_w
        pltpu.sync_copy(idx_hbm.at[pl.ds(base, b_per_w)], idx_v)
        pltpu.async_copy(table_hbm.at[idx_v], rows_v, sem).wait()
        pltpu.sync_copy(rows_v, out_hbm.at[pl.ds(base, b_per_w)])

    return gather(table, idx)


def kernel(x, embeddings):
    en, ent, s = _prep(embeddings)
    nh = N_TOK // 2
    idx3_a, sum_a = _distance_argmin(x[:nh], ent, s)
    idx_a = idx3_a.reshape(nh)
    q_a = _sc_gather(en, idx_a)
    idx3_b, sum_b = _distance_argmin(x[nh:], ent, s)
    idx_b = idx3_b.reshape(nh)
    q_b = _sc_gather(en, idx_b)
    quantized = jnp.concatenate([q_a, q_b], axis=0)
    indices = jnp.concatenate([idx_a, idx_b], axis=0)
    loss = (sum_a[0, 0] + sum_b[0, 0]) * (COMMIT_SCALE / (N_TOK * DIM))
    return (quantized, loss, indices)


# speedup vs baseline: 1.1062x; 1.0874x over previous
import functools

import jax
import jax.numpy as jnp
from jax import lax
from jax.experimental import pallas as pl
from jax.experimental.pallas import tpu as pltpu
from jax.experimental.pallas import tpu_sc as plsc

N_TOK = 36864
DIM = 64
K_EMB = 8192
COMMIT_SCALE = 1.25

BN = 1024


def _tc_body(x_ref, emb_ref, idx_ref, loss_ref, en_ref, en_scr, s_scr, acc_ref):
    i = pl.program_id(0)
    nblk = pl.num_programs(0)

    @pl.when(i == 0)
    def _init():
        e = emb_ref[...]
        en = e / jnp.maximum(
            jnp.sqrt(jnp.sum(e * e, axis=1, keepdims=True)), 1e-12)
        en_scr[...] = en.T
        en_ref[...] = en
        s_scr[...] = jnp.sum(en * en, axis=1)[None, :]
        acc_ref[0] = 0.0

    xb = x_ref[...]
    xn = xb / jnp.maximum(
        jnp.sqrt(jnp.sum(xb * xb, axis=1, keepdims=True)), 1e-12)
    c = jnp.sum(xn * xn, axis=1, keepdims=True)
    dot = lax.dot_general(xn, en_scr[...], (((1,), (0,)), ((), ())),
                          preferred_element_type=jnp.float32)
    dist = c + s_scr[...] - 2.0 * dot
    m = jnp.min(dist, axis=1, keepdims=True)
    kiota = lax.broadcasted_iota(jnp.int32, dist.shape, 1)
    idx_ref[0, 0, :] = jnp.min(jnp.where(dist == m, kiota, K_EMB), axis=1)
    acc_ref[0] += jnp.sum(m)

    @pl.when(i == nblk - 1)
    def _fin():
        loss_ref[0, 0] = acc_ref[0] * (COMMIT_SCALE / (N_TOK * DIM))


def _distance_argmin(x, embeddings):
    nblk = N_TOK // BN
    return pl.pallas_call(
        _tc_body,
        grid=(nblk,),
        in_specs=[
            pl.BlockSpec((BN, DIM), lambda i: (i, 0)),
            pl.BlockSpec((K_EMB, DIM), lambda i: (0, 0)),
        ],
        out_specs=[
            pl.BlockSpec((1, 1, BN), lambda i: (i, 0, 0)),
            pl.BlockSpec(memory_space=pltpu.SMEM),
            pl.BlockSpec((K_EMB, DIM), lambda i: (0, 0)),
        ],
        out_shape=[
            jax.ShapeDtypeStruct((nblk, 1, BN), jnp.int32),
            jax.ShapeDtypeStruct((1, 1), jnp.float32),
            jax.ShapeDtypeStruct((K_EMB, DIM), jnp.float32),
        ],
        scratch_shapes=[
            pltpu.VMEM((DIM, K_EMB), jnp.float32),
            pltpu.VMEM((1, K_EMB), jnp.float32),
            pltpu.SMEM((1,), jnp.float32),
        ],
        compiler_params=pltpu.CompilerParams(
            vmem_limit_bytes=110 * 1024 * 1024),
    )(x, embeddings)


def _sc_gather(table, idx):
    info = plsc.get_sparse_core_info()
    nw = info.num_cores * info.num_subcores
    b_per_w = N_TOK // nw
    mesh = plsc.VectorSubcoreMesh(core_axis_name="c", subcore_axis_name="s")

    @functools.partial(
        pl.kernel,
        out_type=jax.ShapeDtypeStruct((N_TOK, DIM), jnp.float32),
        mesh=mesh,
        compiler_params=pltpu.CompilerParams(use_tc_tiling_on_sc=False),
        scratch_types=[
            pltpu.VMEM((b_per_w,), jnp.int32),
            pltpu.VMEM((b_per_w, DIM), jnp.float32),
            pltpu.SemaphoreType.DMA,
        ],
    )
    def gather(table_hbm, idx_hbm, out_hbm, idx_v, rows_v, sem):
        wid = lax.axis_index("s") * info.num_cores + lax.axis_index("c")
        base = wid * b_per_w
        pltpu.sync_copy(idx_hbm.at[pl.ds(base, b_per_w)], idx_v)
        pltpu.async_copy(table_hbm.at[idx_v], rows_v, sem).wait()
        pltpu.sync_copy(rows_v, out_hbm.at[pl.ds(base, b_per_w)])

    return gather(table, idx)


def kernel(x, embeddings):
    idx3, loss2, e_n = _distance_argmin(x, embeddings)
    indices = idx3.reshape(N_TOK)
    quantized = _sc_gather(e_n, indices)
    return (quantized, loss2[0, 0], indices)
